# Initial kernel scaffold; baseline (speedup 1.0000x reference)
import functools

import jax
import jax.numpy as jnp
from jax import lax
from jax.experimental import pallas as pl
from jax.experimental.pallas import tpu as pltpu
from jax.experimental.pallas import tpu_sc as plsc

C = 64
K = 16
HEAD_C = 16
NEG = 0.2
ONE_MINUS_NEG = 0.8
EPS = 1e-6
NORM_EPS = 1e-12

NB_PREP = 512
RB_KNN = 512
MB_ATT = 64
SC_CHUNK = 128


def _vn_leaky(p_list, d_list):
    dot = sum(p * d for p, d in zip(p_list, d_list))
    dsq = sum(d * d for d in d_list)
    fac = jnp.where(dot < 0, ONE_MINUS_NEG * dot / (dsq + EPS), 0.0)
    return [p - fac * d for p, d in zip(p_list, d_list)]


def _mm(a, w):
    return lax.dot_general(a, w, (((1,), (1,)), ((), ())),
                           preferred_element_type=jnp.float32)


def _prep_body(x_ref, y_ref, wqf_ref, wqd_ref, wkcf_ref, wkcd_ref,
               wvcf_ref, wvcd_ref,
               qx_ref, ckp_ref, ckd_ref, cvp_ref, cvd_ref):
    wqf, wqd = wqf_ref[...], wqd_ref[...]
    qp = [_mm(x_ref[0, d], wqf) for d in range(3)]
    qd = [_mm(x_ref[0, d], wqd) for d in range(3)]
    q = _vn_leaky(qp, qd)
    qn2 = sum(t * t for t in q)
    qn = jnp.sqrt(qn2)
    tot = jnp.sqrt(jnp.sum(qn2, axis=1, keepdims=True))
    scale = (qn / jnp.maximum(tot, NORM_EPS)) / jnp.maximum(qn, NORM_EPS)
    for d in range(3):
        qx_ref[0, d] = q[d] * scale
    for w_ref, o_ref in ((wkcf_ref, ckp_ref), (wkcd_ref, ckd_ref),
                         (wvcf_ref, cvp_ref), (wvcd_ref, cvd_ref)):
        w = w_ref[...]
        for d in range(3):
            o_ref[0, d] = _mm(y_ref[0, d], w)


def _run_prep(x3, y3, wqf, wqd, wkcf, wkcd, wvcf, wvcd):
    B, _, N, _ = x3.shape
    grid = (B, N // NB_PREP)
    blk = pl.BlockSpec((1, 3, NB_PREP, C), lambda b, i: (b, 0, i, 0))
    wblk = pl.BlockSpec((C, C), lambda b, i: (0, 0))
    out = jax.ShapeDtypeStruct((B, 3, N, C), jnp.float32)
    return pl.pallas_call(
        _prep_body,
        grid=grid,
        in_specs=[blk, blk, wblk, wblk, wblk, wblk, wblk, wblk],
        out_specs=[blk] * 5,
        out_shape=[out] * 5,
    )(x3, y3, wqf, wqd, wkcf, wkcd, wvcf, wvcd)


def _knn_body(yr_ref, yc_ref, idx_ref, *, n):
    yr = yr_ref[0]
    yc = yc_ref[0]
    g = lax.dot_general(yr, yc, (((1,), (0,)), ((), ())),
                        preferred_element_type=jnp.float32)
    s_row = jnp.sum(yr * yr, axis=1, keepdims=True)
    s_col = jnp.sum(yc * yc, axis=0, keepdims=True)
    vals = (2.0 * g - s_col) - s_row
    col = lax.broadcasted_iota(jnp.int32, vals.shape, 1)
    picks = []
    for _ in range(K):
        m = jnp.max(vals, axis=1, keepdims=True)
        am = jnp.min(jnp.where(vals == m, col, n), axis=1, keepdims=True)
        picks.append(am)
        vals = jnp.where(col == am, -jnp.inf, vals)
    base = pl.program_id(0) * n
    idx_ref[0] = jnp.concatenate(picks, axis=1) + base


def _run_knn(y_rows, y_cols):
    B, N, _ = y_rows.shape
    grid = (B, N // RB_KNN)
    return pl.pallas_call(
        functools.partial(_knn_body, n=N),
        grid=grid,
        in_specs=[
            pl.BlockSpec((1, RB_KNN, 3 * C), lambda b, i: (b, i, 0)),
            pl.BlockSpec((1, 3 * C, N), lambda b, i: (b, 0, 0)),
        ],
        out_specs=pl.BlockSpec((1, RB_KNN, K), lambda b, i: (b, i, 0)),
        out_shape=jax.ShapeDtypeStruct((B, N, K), jnp.int32),
    )(y_rows, y_cols)


def _sc_gather_body(nc, n_chunks, t0, t1, t2, idx_hbm, g0, g1, g2,
                    idx_v, r0, r1, r2, sem0, sem1, sem2):
    wid = lax.axis_index("s") * nc + lax.axis_index("c")
    base = wid * (n_chunks * SC_CHUNK)

    def chunk(i, carry):
        off = pl.multiple_of(base + i * SC_CHUNK, SC_CHUNK)
        pltpu.sync_copy(idx_hbm.at[pl.ds(off, SC_CHUNK)], idx_v)
        cp0 = pltpu.async_copy(t0.at[idx_v], r0, sem0)
        cp1 = pltpu.async_copy(t1.at[idx_v], r1, sem1)
        cp2 = pltpu.async_copy(t2.at[idx_v], r2, sem2)
        cp0.wait()
        pltpu.sync_copy(r0, g0.at[pl.ds(off, SC_CHUNK)])
        cp1.wait()
        pltpu.sync_copy(r1, g1.at[pl.ds(off, SC_CHUNK)])
        cp2.wait()
        pltpu.sync_copy(r2, g2.at[pl.ds(off, SC_CHUNK)])
        return carry

    lax.fori_loop(0, n_chunks, chunk, 0)


def _run_sc_gather(t0, t1, t2, idx_flat):
    (p,) = idx_flat.shape
    info = plsc.get_sparse_core_info()
    nc, ns = info.num_cores, info.num_subcores
    nw = nc * ns
    n_chunks = p // (nw * SC_CHUNK)
    mesh = plsc.VectorSubcoreMesh(core_axis_name="c", subcore_axis_name="s")
    out = jax.ShapeDtypeStruct((p, C), jnp.float32)
    run = pl.kernel(
        functools.partial(_sc_gather_body, nc, n_chunks),
        out_type=[out] * 3,
        mesh=mesh,
        scratch_types=[
            pltpu.VMEM((SC_CHUNK,), jnp.int32),
            pltpu.VMEM((SC_CHUNK, C), jnp.float32),
            pltpu.VMEM((SC_CHUNK, C), jnp.float32),
            pltpu.VMEM((SC_CHUNK, C), jnp.float32),
            pltpu.SemaphoreType.DMA,
            pltpu.SemaphoreType.DMA,
            pltpu.SemaphoreType.DMA,
        ],
    )
    return run(t0, t1, t2, idx_flat)


def _attend_body(g0_ref, g1_ref, g2_ref, ckp_ref, ckd_ref, cvp_ref,
                 cvd_ref, qx_ref, x_ref, wkaf_ref, wkad_ref, wvaf_ref,
                 wvad_ref, out_ref):
    g_refs = (g0_ref, g1_ref, g2_ref)
    wkaf, wkad = wkaf_ref[...], wkad_ref[...]
    wvaf, wvad = wvaf_ref[...], wvad_ref[...]
    pk, dk, pv, dv = [], [], [], []
    for d in range(3):
        gm = g_refs[d][0, 0].reshape(MB_ATT * K, C)
        pk.append(_mm(gm, wkaf).reshape(MB_ATT, K, C)
                  + ckp_ref[0, d].reshape(MB_ATT, 1, C))
        dk.append(_mm(gm, wkad).reshape(MB_ATT, K, C)
                  + ckd_ref[0, d].reshape(MB_ATT, 1, C))
        pv.append(_mm(gm, wvaf).reshape(MB_ATT, K, C)
                  + cvp_ref[0, d].reshape(MB_ATT, 1, C))
        dv.append(_mm(gm, wvad).reshape(MB_ATT, K, C)
                  + cvd_ref[0, d].reshape(MB_ATT, 1, C))
    kf = _vn_leaky(pk, dk)
    vf = _vn_leaky(pv, dv)
    qn2 = sum(t * t for t in kf)
    qn = jnp.sqrt(qn2)
    tot = jnp.sqrt(jnp.sum(qn2, axis=2, keepdims=True))
    scale = (qn / jnp.maximum(tot, NORM_EPS)) / jnp.maximum(qn, NORM_EPS)
    qk = sum((kf[d] * scale) * qx_ref[0, d].reshape(MB_ATT, 1, C)
             for d in range(3))
    hi = lax.broadcasted_iota(jnp.int32, (C, C), 0) // HEAD_C
    hj = lax.broadcasted_iota(jnp.int32, (C, C), 1) // HEAD_C
    h = (hi == hj).astype(jnp.float32)
    s = lax.dot_general(qk.reshape(MB_ATT * K, C), h,
                        (((1,), (0,)), ((), ())),
                        preferred_element_type=jnp.float32)
    s = s.reshape(MB_ATT, K, C) / jnp.sqrt(jnp.float32(3 * HEAD_C))
    mx = jnp.max(s, axis=1, keepdims=True)
    e = jnp.exp(s - mx)
    a = e / jnp.sum(e, axis=1, keepdims=True)
    for d in range(3):
        out_ref[0, d] = x_ref[0, d] + jnp.sum(a * vf[d], axis=1)


def _run_attend(g0, g1, g2, ckp, ckd, cvp, cvd, qx3, x3,
                wkaf, wkad, wvaf, wvad):
    B, _, N, _ = x3.shape
    nblk = N // MB_ATT
    grid = (B, nblk)
    gshape = (B, nblk, MB_ATT, K, C)
    gspec = pl.BlockSpec((1, 1, MB_ATT, K, C), lambda b, i: (b, i, 0, 0, 0))
    blk = pl.BlockSpec((1, 3, MB_ATT, C), lambda b, i: (b, 0, i, 0))
    wblk = pl.BlockSpec((C, C), lambda b, i: (0, 0))
    return pl.pallas_call(
        _attend_body,
        grid=grid,
        in_specs=[gspec, gspec, gspec, blk, blk, blk, blk, blk, blk,
                  wblk, wblk, wblk, wblk],
        out_specs=blk,
        out_shape=jax.ShapeDtypeStruct((B, 3, N, C), jnp.float32),
    )(g0.reshape(gshape), g1.reshape(gshape), g2.reshape(gshape),
      ckp, ckd, cvp, cvd, qx3, x3, wkaf, wkad, wvaf, wvad)


def kernel(x, y, wq_feat, wq_dir, wk_feat, wk_dir, wv_feat, wv_dir):
    B, Cc, _, N = x.shape
    x3 = jnp.transpose(x, (0, 2, 3, 1))
    y3 = jnp.transpose(y, (0, 2, 3, 1))
    y_rows = jnp.transpose(y, (0, 3, 2, 1)).reshape(B, N, 3 * Cc)
    y_cols = jnp.transpose(y, (0, 2, 1, 3)).reshape(B, 3 * Cc, N)

    wka_f, wkc_f = wk_feat[:, :Cc], wk_feat[:, Cc:] - wk_feat[:, :Cc]
    wka_d, wkc_d = wk_dir[:, :Cc], wk_dir[:, Cc:] - wk_dir[:, :Cc]
    wva_f, wvc_f = wv_feat[:, :Cc], wv_feat[:, Cc:] - wv_feat[:, :Cc]
    wva_d, wvc_d = wv_dir[:, :Cc], wv_dir[:, Cc:] - wv_dir[:, :Cc]

    qx3, ckp, ckd, cvp, cvd = _run_prep(
        x3, y3, wq_feat, wq_dir, wkc_f, wkc_d, wvc_f, wvc_d)

    gidx = _run_knn(y_rows, y_cols)

    t0 = y3[:, 0].reshape(B * N, Cc)
    t1 = y3[:, 1].reshape(B * N, Cc)
    t2 = y3[:, 2].reshape(B * N, Cc)
    g0, g1, g2 = _run_sc_gather(t0, t1, t2, gidx.reshape(-1))

    out3 = _run_attend(g0, g1, g2, ckp, ckd, cvp, cvd, qx3, x3,
                       wka_f, wka_d, wva_f, wva_d)
    return jnp.transpose(out3, (0, 3, 1, 2))


# speedup vs baseline: 7.4022x; 7.4022x over previous
import functools

import jax
import jax.numpy as jnp
from jax import lax
from jax.experimental import pallas as pl
from jax.experimental.pallas import tpu as pltpu
from jax.experimental.pallas import tpu_sc as plsc

C = 64
K = 16
HEAD_C = 16
NEG = 0.2
ONE_MINUS_NEG = 0.8
EPS = 1e-6
NORM_EPS = 1e-12

NB_PREP = 512
RB_KNN = 512
MB_ATT = 64
SC_CHUNK = 128


def _vn_leaky(p_list, d_list):
    dot = sum(p * d for p, d in zip(p_list, d_list))
    dsq = sum(d * d for d in d_list)
    fac = jnp.where(dot < 0, ONE_MINUS_NEG * dot / (dsq + EPS), 0.0)
    return [p - fac * d for p, d in zip(p_list, d_list)]


def _mm(a, w):
    return lax.dot_general(a, w, (((1,), (1,)), ((), ())),
                           preferred_element_type=jnp.float32)


def _prep_body(x_ref, y_ref, wqf_ref, wqd_ref, wkcf_ref, wkcd_ref,
               wvcf_ref, wvcd_ref,
               qx_ref, ckp_ref, ckd_ref, cvp_ref, cvd_ref):
    wqf, wqd = wqf_ref[...], wqd_ref[...]
    qp = [_mm(x_ref[0, d], wqf) for d in range(3)]
    qd = [_mm(x_ref[0, d], wqd) for d in range(3)]
    q = _vn_leaky(qp, qd)
    qn2 = sum(t * t for t in q)
    qn = jnp.sqrt(qn2)
    tot = jnp.sqrt(jnp.sum(qn2, axis=1, keepdims=True))
    scale = (qn / jnp.maximum(tot, NORM_EPS)) / jnp.maximum(qn, NORM_EPS)
    for d in range(3):
        qx_ref[0, d] = q[d] * scale
    for w_ref, o_ref in ((wkcf_ref, ckp_ref), (wkcd_ref, ckd_ref),
                         (wvcf_ref, cvp_ref), (wvcd_ref, cvd_ref)):
        w = w_ref[...]
        for d in range(3):
            o_ref[0, d] = _mm(y_ref[0, d], w)


def _run_prep(x3, y3, wqf, wqd, wkcf, wkcd, wvcf, wvcd):
    B, _, N, _ = x3.shape
    grid = (B, N // NB_PREP)
    blk = pl.BlockSpec((1, 3, NB_PREP, C), lambda b, i: (b, 0, i, 0))
    wblk = pl.BlockSpec((C, C), lambda b, i: (0, 0))
    out = jax.ShapeDtypeStruct((B, 3, N, C), jnp.float32)
    return pl.pallas_call(
        _prep_body,
        grid=grid,
        in_specs=[blk, blk, wblk, wblk, wblk, wblk, wblk, wblk],
        out_specs=[blk] * 5,
        out_shape=[out] * 5,
    )(x3, y3, wqf, wqd, wkcf, wkcd, wvcf, wvcd)


def _knn_body(yr_ref, yc_ref, idx_ref, *, n):
    yr = yr_ref[0]
    yc = yc_ref[0]
    g = lax.dot_general(yr, yc, (((1,), (0,)), ((), ())),
                        preferred_element_type=jnp.float32)
    s_row = jnp.sum(yr * yr, axis=1, keepdims=True)
    s_col = jnp.sum(yc * yc, axis=0, keepdims=True)
    vals = (2.0 * g - s_col) - s_row
    col = lax.broadcasted_iota(jnp.int32, vals.shape, 1)
    picks = []
    for _ in range(K):
        m = jnp.max(vals, axis=1, keepdims=True)
        am = jnp.min(jnp.where(vals == m, col, n), axis=1, keepdims=True)
        picks.append(am)
        vals = jnp.where(col == am, -jnp.inf, vals)
    base = pl.program_id(0) * n
    idx_ref[0] = jnp.concatenate(picks, axis=1) + base


def _run_knn(y_rows, y_cols):
    B, N, _ = y_rows.shape
    grid = (B, N // RB_KNN)
    return pl.pallas_call(
        functools.partial(_knn_body, n=N),
        grid=grid,
        in_specs=[
            pl.BlockSpec((1, RB_KNN, 3 * C), lambda b, i: (b, i, 0)),
            pl.BlockSpec((1, 3 * C, N), lambda b, i: (b, 0, 0)),
        ],
        out_specs=pl.BlockSpec((1, RB_KNN, K), lambda b, i: (b, i, 0)),
        out_shape=jax.ShapeDtypeStruct((B, N, K), jnp.int32),
    )(y_rows, y_cols)


def _sc_gather_body(nc, n_chunks, t0, t1, t2, idx_hbm, g0, g1, g2,
                    idx_v, r0, r1, r2, sem0, sem1, sem2):
    wid = lax.axis_index("s") * nc + lax.axis_index("c")
    base = wid * (n_chunks * SC_CHUNK)

    def chunk(i, carry):
        off = pl.multiple_of(base + i * SC_CHUNK, SC_CHUNK)
        pltpu.sync_copy(idx_hbm.at[pl.ds(off, SC_CHUNK)], idx_v)
        cp0 = pltpu.async_copy(t0.at[idx_v], r0, sem0)
        cp1 = pltpu.async_copy(t1.at[idx_v], r1, sem1)
        cp2 = pltpu.async_copy(t2.at[idx_v], r2, sem2)
        cp0.wait()
        pltpu.sync_copy(r0, g0.at[pl.ds(off, SC_CHUNK)])
        cp1.wait()
        pltpu.sync_copy(r1, g1.at[pl.ds(off, SC_CHUNK)])
        cp2.wait()
        pltpu.sync_copy(r2, g2.at[pl.ds(off, SC_CHUNK)])
        return carry

    lax.fori_loop(0, n_chunks, chunk, 0)


def _run_sc_gather(t0, t1, t2, idx_flat):
    (p,) = idx_flat.shape
    info = plsc.get_sparse_core_info()
    nc, ns = info.num_cores, info.num_subcores
    nw = nc * ns
    n_chunks = p // (nw * SC_CHUNK)
    mesh = plsc.VectorSubcoreMesh(core_axis_name="c", subcore_axis_name="s")
    out = jax.ShapeDtypeStruct((p, C), jnp.float32)
    run = pl.kernel(
        functools.partial(_sc_gather_body, nc, n_chunks),
        out_type=[out] * 3,
        mesh=mesh,
        compiler_params=pltpu.CompilerParams(use_tc_tiling_on_sc=False),
        scratch_types=[
            pltpu.VMEM((SC_CHUNK,), jnp.int32),
            pltpu.VMEM((SC_CHUNK, C), jnp.float32),
            pltpu.VMEM((SC_CHUNK, C), jnp.float32),
            pltpu.VMEM((SC_CHUNK, C), jnp.float32),
            pltpu.SemaphoreType.DMA,
            pltpu.SemaphoreType.DMA,
            pltpu.SemaphoreType.DMA,
        ],
    )
    return run(t0, t1, t2, idx_flat)


def _attend_body(g0_ref, g1_ref, g2_ref, ckp_ref, ckd_ref, cvp_ref,
                 cvd_ref, qx_ref, x_ref, wkaf_ref, wkad_ref, wvaf_ref,
                 wvad_ref, out_ref):
    g_refs = (g0_ref, g1_ref, g2_ref)
    wkaf, wkad = wkaf_ref[...], wkad_ref[...]
    wvaf, wvad = wvaf_ref[...], wvad_ref[...]
    pk, dk, pv, dv = [], [], [], []
    for d in range(3):
        gm = g_refs[d][0, 0].reshape(MB_ATT * K, C)
        pk.append(_mm(gm, wkaf).reshape(MB_ATT, K, C)
                  + ckp_ref[0, d].reshape(MB_ATT, 1, C))
        dk.append(_mm(gm, wkad).reshape(MB_ATT, K, C)
                  + ckd_ref[0, d].reshape(MB_ATT, 1, C))
        pv.append(_mm(gm, wvaf).reshape(MB_ATT, K, C)
                  + cvp_ref[0, d].reshape(MB_ATT, 1, C))
        dv.append(_mm(gm, wvad).reshape(MB_ATT, K, C)
                  + cvd_ref[0, d].reshape(MB_ATT, 1, C))
    kf = _vn_leaky(pk, dk)
    vf = _vn_leaky(pv, dv)
    qn2 = sum(t * t for t in kf)
    qn = jnp.sqrt(qn2)
    tot = jnp.sqrt(jnp.sum(qn2, axis=2, keepdims=True))
    scale = (qn / jnp.maximum(tot, NORM_EPS)) / jnp.maximum(qn, NORM_EPS)
    qk = sum((kf[d] * scale) * qx_ref[0, d].reshape(MB_ATT, 1, C)
             for d in range(3))
    hi = lax.broadcasted_iota(jnp.int32, (C, C), 0) // HEAD_C
    hj = lax.broadcasted_iota(jnp.int32, (C, C), 1) // HEAD_C
    h = (hi == hj).astype(jnp.float32)
    s = lax.dot_general(qk.reshape(MB_ATT * K, C), h,
                        (((1,), (0,)), ((), ())),
                        preferred_element_type=jnp.float32)
    s = s.reshape(MB_ATT, K, C) / jnp.sqrt(jnp.float32(3 * HEAD_C))
    mx = jnp.max(s, axis=1, keepdims=True)
    e = jnp.exp(s - mx)
    a = e / jnp.sum(e, axis=1, keepdims=True)
    for d in range(3):
        out_ref[0, d] = x_ref[0, d] + jnp.sum(a * vf[d], axis=1)


def _run_attend(g0, g1, g2, ckp, ckd, cvp, cvd, qx3, x3,
                wkaf, wkad, wvaf, wvad):
    B, _, N, _ = x3.shape
    nblk = N // MB_ATT
    grid = (B, nblk)
    gshape = (B, nblk, MB_ATT, K, C)
    gspec = pl.BlockSpec((1, 1, MB_ATT, K, C), lambda b, i: (b, i, 0, 0, 0))
    blk = pl.BlockSpec((1, 3, MB_ATT, C), lambda b, i: (b, 0, i, 0))
    wblk = pl.BlockSpec((C, C), lambda b, i: (0, 0))
    return pl.pallas_call(
        _attend_body,
        grid=grid,
        in_specs=[gspec, gspec, gspec, blk, blk, blk, blk, blk, blk,
                  wblk, wblk, wblk, wblk],
        out_specs=blk,
        out_shape=jax.ShapeDtypeStruct((B, 3, N, C), jnp.float32),
    )(g0.reshape(gshape), g1.reshape(gshape), g2.reshape(gshape),
      ckp, ckd, cvp, cvd, qx3, x3, wkaf, wkad, wvaf, wvad)


def kernel(x, y, wq_feat, wq_dir, wk_feat, wk_dir, wv_feat, wv_dir):
    B, Cc, _, N = x.shape
    x3 = jnp.transpose(x, (0, 2, 3, 1))
    y3 = jnp.transpose(y, (0, 2, 3, 1))
    y_rows = jnp.transpose(y, (0, 3, 2, 1)).reshape(B, N, 3 * Cc)
    y_cols = jnp.transpose(y, (0, 2, 1, 3)).reshape(B, 3 * Cc, N)

    wka_f, wkc_f = wk_feat[:, :Cc], wk_feat[:, Cc:] - wk_feat[:, :Cc]
    wka_d, wkc_d = wk_dir[:, :Cc], wk_dir[:, Cc:] - wk_dir[:, :Cc]
    wva_f, wvc_f = wv_feat[:, :Cc], wv_feat[:, Cc:] - wv_feat[:, :Cc]
    wva_d, wvc_d = wv_dir[:, :Cc], wv_dir[:, Cc:] - wv_dir[:, :Cc]

    qx3, ckp, ckd, cvp, cvd = _run_prep(
        x3, y3, wq_feat, wq_dir, wkc_f, wkc_d, wvc_f, wvc_d)

    gidx = _run_knn(y_rows, y_cols)

    t0 = y3[:, 0].reshape(B * N, Cc)
    t1 = y3[:, 1].reshape(B * N, Cc)
    t2 = y3[:, 2].reshape(B * N, Cc)
    g0, g1, g2 = _run_sc_gather(t0, t1, t2, gidx.reshape(-1))

    out3 = _run_attend(g0, g1, g2, ckp, ckd, cvp, cvd, qx3, x3,
                       wka_f, wka_d, wva_f, wva_d)
    return jnp.transpose(out3, (0, 3, 1, 2))


# speedup vs baseline: 8.3906x; 1.1335x over previous
import functools

import jax
import jax.numpy as jnp
from jax import lax
from jax.experimental import pallas as pl
from jax.experimental.pallas import tpu as pltpu
from jax.experimental.pallas import tpu_sc as plsc

C = 64
K = 16
HEAD_C = 16
NEG = 0.2
ONE_MINUS_NEG = 0.8
EPS = 1e-6
NORM_EPS = 1e-12

NB_PREP = 512
RB_KNN = 512
MB_ATT = 64
SC_CHUNK = 128


def _vn_leaky(p_list, d_list):
    dot = sum(p * d for p, d in zip(p_list, d_list))
    dsq = sum(d * d for d in d_list)
    fac = jnp.where(dot < 0, ONE_MINUS_NEG * dot / (dsq + EPS), 0.0)
    return [p - fac * d for p, d in zip(p_list, d_list)]


def _mm(a, w):
    return lax.dot_general(a, w, (((1,), (1,)), ((), ())),
                           preferred_element_type=jnp.float32)


def _prep_body(x_ref, y_ref, wqf_ref, wqd_ref, wkcf_ref, wkcd_ref,
               wvcf_ref, wvcd_ref,
               qx_ref, ckp_ref, ckd_ref, cvp_ref, cvd_ref):
    wqf, wqd = wqf_ref[...], wqd_ref[...]
    qp = [_mm(x_ref[0, d], wqf) for d in range(3)]
    qd = [_mm(x_ref[0, d], wqd) for d in range(3)]
    q = _vn_leaky(qp, qd)
    qn2 = sum(t * t for t in q)
    tot = jnp.sqrt(jnp.sum(qn2, axis=1, keepdims=True))
    inv_tot = 1.0 / jnp.maximum(tot, NORM_EPS)
    for d in range(3):
        qx_ref[0, d] = q[d] * inv_tot
    for w_ref, o_ref in ((wkcf_ref, ckp_ref), (wkcd_ref, ckd_ref),
                         (wvcf_ref, cvp_ref), (wvcd_ref, cvd_ref)):
        w = w_ref[...]
        for d in range(3):
            o_ref[0, d] = _mm(y_ref[0, d], w)


def _run_prep(x3, y3, wqf, wqd, wkcf, wkcd, wvcf, wvcd):
    B, _, N, _ = x3.shape
    grid = (B, N // NB_PREP)
    blk = pl.BlockSpec((1, 3, NB_PREP, C), lambda b, i: (b, 0, i, 0))
    wblk = pl.BlockSpec((C, C), lambda b, i: (0, 0))
    out = jax.ShapeDtypeStruct((B, 3, N, C), jnp.float32)
    return pl.pallas_call(
        _prep_body,
        grid=grid,
        in_specs=[blk, blk, wblk, wblk, wblk, wblk, wblk, wblk],
        out_specs=[blk] * 5,
        out_shape=[out] * 5,
    )(x3, y3, wqf, wqd, wkcf, wkcd, wvcf, wvcd)


def _knn_body(yr_ref, yc_ref, idx_ref, *, n):
    yr = yr_ref[0]
    yc = yc_ref[0]
    g = lax.dot_general(yr, yc, (((1,), (0,)), ((), ())),
                        preferred_element_type=jnp.float32)
    s_row = jnp.sum(yr * yr, axis=1, keepdims=True)
    s_col = jnp.sum(yc * yc, axis=0, keepdims=True)
    vals = (2.0 * g - s_col) - s_row
    u = lax.bitcast_convert_type(vals, jnp.int32)
    ki = jnp.where(u >= 0, u, u ^ jnp.int32(0x7FFFFFFF))
    col = lax.broadcasted_iota(jnp.int32, vals.shape, 1)
    keys = (ki & jnp.int32(~2047)) | (jnp.int32(n - 1) - col)
    picks = []
    int_min = jnp.int32(-(2 ** 31))
    for _ in range(K):
        m = jnp.max(keys, axis=1, keepdims=True)
        picks.append(jnp.int32(n - 1) - (m & jnp.int32(2047)))
        keys = jnp.where(keys == m, int_min, keys)
    base = pl.program_id(0) * n
    idx_ref[0] = jnp.concatenate(picks, axis=1) + base


def _run_knn(y_rows, y_cols):
    B, N, _ = y_rows.shape
    grid = (B, N // RB_KNN)
    return pl.pallas_call(
        functools.partial(_knn_body, n=N),
        grid=grid,
        in_specs=[
            pl.BlockSpec((1, RB_KNN, 3 * C), lambda b, i: (b, i, 0)),
            pl.BlockSpec((1, 3 * C, N), lambda b, i: (b, 0, 0)),
        ],
        out_specs=pl.BlockSpec((1, RB_KNN, K), lambda b, i: (b, i, 0)),
        out_shape=jax.ShapeDtypeStruct((B, N, K), jnp.int32),
    )(y_rows, y_cols)


def _sc_gather_body(nc, n_chunks, t0, t1, t2, idx_hbm, g0, g1, g2,
                    idx_v, r0, r1, r2, sem0, sem1, sem2):
    wid = lax.axis_index("s") * nc + lax.axis_index("c")
    base = wid * (n_chunks * SC_CHUNK)

    def chunk(i, carry):
        off = pl.multiple_of(base + i * SC_CHUNK, SC_CHUNK)
        pltpu.sync_copy(idx_hbm.at[pl.ds(off, SC_CHUNK)], idx_v)
        cp0 = pltpu.async_copy(t0.at[idx_v], r0, sem0)
        cp1 = pltpu.async_copy(t1.at[idx_v], r1, sem1)
        cp2 = pltpu.async_copy(t2.at[idx_v], r2, sem2)
        cp0.wait()
        pltpu.sync_copy(r0, g0.at[pl.ds(off, SC_CHUNK)])
        cp1.wait()
        pltpu.sync_copy(r1, g1.at[pl.ds(off, SC_CHUNK)])
        cp2.wait()
        pltpu.sync_copy(r2, g2.at[pl.ds(off, SC_CHUNK)])
        return carry

    lax.fori_loop(0, n_chunks, chunk, 0)


def _run_sc_gather(t0, t1, t2, idx_flat):
    (p,) = idx_flat.shape
    info = plsc.get_sparse_core_info()
    nc, ns = info.num_cores, info.num_subcores
    nw = nc * ns
    n_chunks = p // (nw * SC_CHUNK)
    mesh = plsc.VectorSubcoreMesh(core_axis_name="c", subcore_axis_name="s")
    out = jax.ShapeDtypeStruct((p, C), jnp.float32)
    run = pl.kernel(
        functools.partial(_sc_gather_body, nc, n_chunks),
        out_type=[out] * 3,
        mesh=mesh,
        compiler_params=pltpu.CompilerParams(use_tc_tiling_on_sc=False),
        scratch_types=[
            pltpu.VMEM((SC_CHUNK,), jnp.int32),
            pltpu.VMEM((SC_CHUNK, C), jnp.float32),
            pltpu.VMEM((SC_CHUNK, C), jnp.float32),
            pltpu.VMEM((SC_CHUNK, C), jnp.float32),
            pltpu.SemaphoreType.DMA,
            pltpu.SemaphoreType.DMA,
            pltpu.SemaphoreType.DMA,
        ],
    )
    return run(t0, t1, t2, idx_flat)


def _attend_body(g0_ref, g1_ref, g2_ref, ckp_ref, ckd_ref, cvp_ref,
                 cvd_ref, qx_ref, x_ref, wkaf_ref, wkad_ref, wvaf_ref,
                 wvad_ref, out_ref):
    g_refs = (g0_ref, g1_ref, g2_ref)
    wkaf, wkad = wkaf_ref[...], wkad_ref[...]
    wvaf, wvad = wvaf_ref[...], wvad_ref[...]
    pk, dk, pv, dv = [], [], [], []
    for d in range(3):
        gm = g_refs[d][0, 0].reshape(MB_ATT * K, C)
        pk.append(_mm(gm, wkaf).reshape(MB_ATT, K, C)
                  + ckp_ref[0, d].reshape(MB_ATT, 1, C))
        dk.append(_mm(gm, wkad).reshape(MB_ATT, K, C)
                  + ckd_ref[0, d].reshape(MB_ATT, 1, C))
        pv.append(_mm(gm, wvaf).reshape(MB_ATT, K, C)
                  + cvp_ref[0, d].reshape(MB_ATT, 1, C))
        dv.append(_mm(gm, wvad).reshape(MB_ATT, K, C)
                  + cvd_ref[0, d].reshape(MB_ATT, 1, C))
    kf = _vn_leaky(pk, dk)
    vf = _vn_leaky(pv, dv)
    qn2 = sum(t * t for t in kf)
    tot = jnp.sqrt(jnp.sum(qn2, axis=2, keepdims=True))
    inv_tot = 1.0 / jnp.maximum(tot, NORM_EPS)
    qk = sum(kf[d] * qx_ref[0, d].reshape(MB_ATT, 1, C)
             for d in range(3)) * inv_tot
    hi = lax.broadcasted_iota(jnp.int32, (C, C), 0) // HEAD_C
    hj = lax.broadcasted_iota(jnp.int32, (C, C), 1) // HEAD_C
    h = (hi == hj).astype(jnp.float32) / jnp.sqrt(jnp.float32(3 * HEAD_C))
    s = lax.dot_general(qk.reshape(MB_ATT * K, C), h,
                        (((1,), (0,)), ((), ())),
                        preferred_element_type=jnp.float32)
    s = s.reshape(MB_ATT, K, C)
    mx = jnp.max(s, axis=1, keepdims=True)
    e = jnp.exp(s - mx)
    r = 1.0 / jnp.sum(e, axis=1)
    for d in range(3):
        out_ref[0, d] = x_ref[0, d] + jnp.sum(e * vf[d], axis=1) * r


def _run_attend(g0, g1, g2, ckp, ckd, cvp, cvd, qx3, x3,
                wkaf, wkad, wvaf, wvad):
    B, _, N, _ = x3.shape
    nblk = N // MB_ATT
    grid = (B, nblk)
    gshape = (B, nblk, MB_ATT, K, C)
    gspec = pl.BlockSpec((1, 1, MB_ATT, K, C), lambda b, i: (b, i, 0, 0, 0))
    blk = pl.BlockSpec((1, 3, MB_ATT, C), lambda b, i: (b, 0, i, 0))
    wblk = pl.BlockSpec((C, C), lambda b, i: (0, 0))
    return pl.pallas_call(
        _attend_body,
        grid=grid,
        in_specs=[gspec, gspec, gspec, blk, blk, blk, blk, blk, blk,
                  wblk, wblk, wblk, wblk],
        out_specs=blk,
        out_shape=jax.ShapeDtypeStruct((B, 3, N, C), jnp.float32),
    )(g0.reshape(gshape), g1.reshape(gshape), g2.reshape(gshape),
      ckp, ckd, cvp, cvd, qx3, x3, wkaf, wkad, wvaf, wvad)


def kernel(x, y, wq_feat, wq_dir, wk_feat, wk_dir, wv_feat, wv_dir):
    B, Cc, _, N = x.shape
    x3 = jnp.transpose(x, (0, 2, 3, 1))
    y3 = jnp.transpose(y, (0, 2, 3, 1))
    y_rows = jnp.transpose(y, (0, 3, 2, 1)).reshape(B, N, 3 * Cc)
    y_cols = jnp.transpose(y, (0, 2, 1, 3)).reshape(B, 3 * Cc, N)

    wka_f, wkc_f = wk_feat[:, :Cc], wk_feat[:, Cc:] - wk_feat[:, :Cc]
    wka_d, wkc_d = wk_dir[:, :Cc], wk_dir[:, Cc:] - wk_dir[:, :Cc]
    wva_f, wvc_f = wv_feat[:, :Cc], wv_feat[:, Cc:] - wv_feat[:, :Cc]
    wva_d, wvc_d = wv_dir[:, :Cc], wv_dir[:, Cc:] - wv_dir[:, :Cc]

    qx3, ckp, ckd, cvp, cvd = _run_prep(
        x3, y3, wq_feat, wq_dir, wkc_f, wkc_d, wvc_f, wvc_d)

    gidx = _run_knn(y_rows, y_cols)

    t0 = y3[:, 0].reshape(B * N, Cc)
    t1 = y3[:, 1].reshape(B * N, Cc)
    t2 = y3[:, 2].reshape(B * N, Cc)
    g0, g1, g2 = _run_sc_gather(t0, t1, t2, gidx.reshape(-1))

    out3 = _run_attend(g0, g1, g2, ckp, ckd, cvp, cvd, qx3, x3,
                       wka_f, wka_d, wva_f, wva_d)
    return jnp.transpose(out3, (0, 3, 1, 2))


# speedup vs baseline: 9.0421x; 1.0776x over previous
import functools

import jax
import jax.numpy as jnp
from jax import lax
from jax.experimental import pallas as pl
from jax.experimental.pallas import tpu as pltpu
from jax.experimental.pallas import tpu_sc as plsc

C = 64
K = 16
HEAD_C = 16
NEG = 0.2
ONE_MINUS_NEG = 0.8
EPS = 1e-6
NORM_EPS = 1e-12

NB_PREP = 512
RB_KNN = 512
MB_ATT = 128
SC_CHUNK = 128


def _vn_leaky(p_list, d_list):
    dot = sum(p * d for p, d in zip(p_list, d_list))
    dsq = sum(d * d for d in d_list)
    fac = jnp.where(dot < 0, ONE_MINUS_NEG * dot / (dsq + EPS), 0.0)
    return [p - fac * d for p, d in zip(p_list, d_list)]


def _mm(a, w):
    return lax.dot_general(a, w, (((1,), (1,)), ((), ())),
                           preferred_element_type=jnp.float32)


def _prep_body(x_ref, y_ref, wqf_ref, wqd_ref, wkcf_ref, wkcd_ref,
               wvcf_ref, wvcd_ref,
               qx_ref, ckp_ref, ckd_ref, cvp_ref, cvd_ref):
    wqf, wqd = wqf_ref[...], wqd_ref[...]
    qp = [_mm(x_ref[0, d], wqf) for d in range(3)]
    qd = [_mm(x_ref[0, d], wqd) for d in range(3)]
    q = _vn_leaky(qp, qd)
    qn2 = sum(t * t for t in q)
    tot = jnp.sqrt(jnp.sum(qn2, axis=1, keepdims=True))
    inv_tot = 1.0 / jnp.maximum(tot, NORM_EPS)
    for d in range(3):
        qx_ref[0, d] = q[d] * inv_tot
    for w_ref, o_ref in ((wkcf_ref, ckp_ref), (wkcd_ref, ckd_ref),
                         (wvcf_ref, cvp_ref), (wvcd_ref, cvd_ref)):
        w = w_ref[...]
        for d in range(3):
            o_ref[0, d] = _mm(y_ref[0, d], w)


def _run_prep(x3, y3, wqf, wqd, wkcf, wkcd, wvcf, wvcd):
    B, _, N, _ = x3.shape
    grid = (B, N // NB_PREP)
    blk = pl.BlockSpec((1, 3, NB_PREP, C), lambda b, i: (b, 0, i, 0))
    wblk = pl.BlockSpec((C, C), lambda b, i: (0, 0))
    out = jax.ShapeDtypeStruct((B, 3, N, C), jnp.float32)
    return pl.pallas_call(
        _prep_body,
        grid=grid,
        in_specs=[blk, blk, wblk, wblk, wblk, wblk, wblk, wblk],
        out_specs=[blk] * 5,
        out_shape=[out] * 5,
    )(x3, y3, wqf, wqd, wkcf, wkcd, wvcf, wvcd)


def _knn_body(yr_ref, yc_ref, idx_ref, *, n):
    yr = yr_ref[0]
    yc = yc_ref[0]
    g = lax.dot_general(yr, yc, (((1,), (0,)), ((), ())),
                        preferred_element_type=jnp.float32)
    s_row = jnp.sum(yr * yr, axis=1, keepdims=True)
    s_col = jnp.sum(yc * yc, axis=0, keepdims=True)
    vals = (2.0 * g - s_col) - s_row
    u = lax.bitcast_convert_type(vals, jnp.int32)
    ki = jnp.where(u >= 0, u, u ^ jnp.int32(0x7FFFFFFF))
    col = lax.broadcasted_iota(jnp.int32, vals.shape, 1)
    keys = (ki & jnp.int32(~2047)) | (jnp.int32(n - 1) - col)
    picks = []
    int_min = jnp.int32(-(2 ** 31))
    for _ in range(K):
        m = jnp.max(keys, axis=1, keepdims=True)
        picks.append(jnp.int32(n - 1) - (m & jnp.int32(2047)))
        keys = jnp.where(keys == m, int_min, keys)
    base = pl.program_id(0) * n
    idx_ref[0] = jnp.concatenate(picks, axis=1) + base


def _run_knn(y_rows, y_cols):
    B, N, _ = y_rows.shape
    grid = (B, N // RB_KNN)
    return pl.pallas_call(
        functools.partial(_knn_body, n=N),
        grid=grid,
        in_specs=[
            pl.BlockSpec((1, RB_KNN, 3 * C), lambda b, i: (b, i, 0)),
            pl.BlockSpec((1, 3 * C, N), lambda b, i: (b, 0, 0)),
        ],
        out_specs=pl.BlockSpec((1, RB_KNN, K), lambda b, i: (b, i, 0)),
        out_shape=jax.ShapeDtypeStruct((B, N, K), jnp.int32),
    )(y_rows, y_cols)


def _sc_gather_body(nc, n_chunks, t0, t1, t2, idx_hbm, g0, g1, g2,
                    idx_v, r0, r1, r2, sem0, sem1, sem2):
    wid = lax.axis_index("s") * nc + lax.axis_index("c")
    base = wid * (n_chunks * SC_CHUNK)

    def chunk(i, carry):
        off = pl.multiple_of(base + i * SC_CHUNK, SC_CHUNK)
        pltpu.sync_copy(idx_hbm.at[pl.ds(off, SC_CHUNK)], idx_v)
        cp0 = pltpu.async_copy(t0.at[idx_v], r0, sem0)
        cp1 = pltpu.async_copy(t1.at[idx_v], r1, sem1)
        cp2 = pltpu.async_copy(t2.at[idx_v], r2, sem2)
        cp0.wait()
        pltpu.sync_copy(r0, g0.at[pl.ds(off, SC_CHUNK)])
        cp1.wait()
        pltpu.sync_copy(r1, g1.at[pl.ds(off, SC_CHUNK)])
        cp2.wait()
        pltpu.sync_copy(r2, g2.at[pl.ds(off, SC_CHUNK)])
        return carry

    lax.fori_loop(0, n_chunks, chunk, 0)


def _run_sc_gather(t0, t1, t2, idx_flat):
    (p,) = idx_flat.shape
    info = plsc.get_sparse_core_info()
    nc, ns = info.num_cores, info.num_subcores
    nw = nc * ns
    n_chunks = p // (nw * SC_CHUNK)
    mesh = plsc.VectorSubcoreMesh(core_axis_name="c", subcore_axis_name="s")
    out = jax.ShapeDtypeStruct((p, C), jnp.float32)
    run = pl.kernel(
        functools.partial(_sc_gather_body, nc, n_chunks),
        out_type=[out] * 3,
        mesh=mesh,
        compiler_params=pltpu.CompilerParams(use_tc_tiling_on_sc=False),
        scratch_types=[
            pltpu.VMEM((SC_CHUNK,), jnp.int32),
            pltpu.VMEM((SC_CHUNK, C), jnp.float32),
            pltpu.VMEM((SC_CHUNK, C), jnp.float32),
            pltpu.VMEM((SC_CHUNK, C), jnp.float32),
            pltpu.SemaphoreType.DMA,
            pltpu.SemaphoreType.DMA,
            pltpu.SemaphoreType.DMA,
        ],
    )
    return run(t0, t1, t2, idx_flat)


def _attend_body(g0_ref, g1_ref, g2_ref, ckp_ref, ckd_ref, cvp_ref,
                 cvd_ref, qx_ref, x_ref, wkaf_ref, wkad_ref, wvaf_ref,
                 wvad_ref, out_ref):
    g_refs = (g0_ref, g1_ref, g2_ref)
    wkaf, wkad = wkaf_ref[...], wkad_ref[...]
    wvaf, wvad = wvaf_ref[...], wvad_ref[...]
    pk, dk, pv, dv = [], [], [], []
    for d in range(3):
        gm = g_refs[d][0, :, 0].reshape(K * MB_ATT, C)
        ckp = ckp_ref[0, d].reshape(1, MB_ATT, C)
        ckd = ckd_ref[0, d].reshape(1, MB_ATT, C)
        cvp = cvp_ref[0, d].reshape(1, MB_ATT, C)
        cvd = cvd_ref[0, d].reshape(1, MB_ATT, C)
        pk.append(_mm(gm, wkaf).reshape(K, MB_ATT, C) + ckp)
        dk.append(_mm(gm, wkad).reshape(K, MB_ATT, C) + ckd)
        pv.append(_mm(gm, wvaf).reshape(K, MB_ATT, C) + cvp)
        dv.append(_mm(gm, wvad).reshape(K, MB_ATT, C) + cvd)
    kf = _vn_leaky(pk, dk)
    vf = _vn_leaky(pv, dv)
    qn2 = sum(t * t for t in kf)
    tot = jnp.sqrt(jnp.sum(qn2, axis=2, keepdims=True))
    inv_tot = 1.0 / jnp.maximum(tot, NORM_EPS)
    qk = sum(kf[d] * qx_ref[0, d].reshape(1, MB_ATT, C)
             for d in range(3)) * inv_tot
    hi = lax.broadcasted_iota(jnp.int32, (C, C), 0) // HEAD_C
    hj = lax.broadcasted_iota(jnp.int32, (C, C), 1) // HEAD_C
    h = (hi == hj).astype(jnp.float32) / jnp.sqrt(jnp.float32(3 * HEAD_C))
    s = lax.dot_general(qk.reshape(K * MB_ATT, C), h,
                        (((1,), (0,)), ((), ())),
                        preferred_element_type=jnp.float32)
    s = s.reshape(K, MB_ATT, C)
    mx = jnp.max(s, axis=0, keepdims=True)
    e = jnp.exp(s - mx)
    r = 1.0 / jnp.sum(e, axis=0)
    for d in range(3):
        out_ref[0, d] = x_ref[0, d] + jnp.sum(e * vf[d], axis=0) * r


def _run_attend(g0, g1, g2, ckp, ckd, cvp, cvd, qx3, x3,
                wkaf, wkad, wvaf, wvad):
    B, _, N, _ = x3.shape
    nblk = N // MB_ATT
    grid = (B, nblk)
    gshape = (B, K, nblk, MB_ATT, C)
    gspec = pl.BlockSpec((1, K, 1, MB_ATT, C), lambda b, i: (b, 0, i, 0, 0))
    blk = pl.BlockSpec((1, 3, MB_ATT, C), lambda b, i: (b, 0, i, 0))
    wblk = pl.BlockSpec((C, C), lambda b, i: (0, 0))
    return pl.pallas_call(
        _attend_body,
        grid=grid,
        in_specs=[gspec, gspec, gspec, blk, blk, blk, blk, blk, blk,
                  wblk, wblk, wblk, wblk],
        out_specs=blk,
        out_shape=jax.ShapeDtypeStruct((B, 3, N, C), jnp.float32),
    )(g0.reshape(gshape), g1.reshape(gshape), g2.reshape(gshape),
      ckp, ckd, cvp, cvd, qx3, x3, wkaf, wkad, wvaf, wvad)


def kernel(x, y, wq_feat, wq_dir, wk_feat, wk_dir, wv_feat, wv_dir):
    B, Cc, _, N = x.shape
    x3 = jnp.transpose(x, (0, 2, 3, 1))
    y3 = jnp.transpose(y, (0, 2, 3, 1))
    y_rows = jnp.transpose(y, (0, 3, 2, 1)).reshape(B, N, 3 * Cc)
    y_cols = jnp.transpose(y, (0, 2, 1, 3)).reshape(B, 3 * Cc, N)

    wka_f, wkc_f = wk_feat[:, :Cc], wk_feat[:, Cc:] - wk_feat[:, :Cc]
    wka_d, wkc_d = wk_dir[:, :Cc], wk_dir[:, Cc:] - wk_dir[:, :Cc]
    wva_f, wvc_f = wv_feat[:, :Cc], wv_feat[:, Cc:] - wv_feat[:, :Cc]
    wva_d, wvc_d = wv_dir[:, :Cc], wv_dir[:, Cc:] - wv_dir[:, :Cc]

    qx3, ckp, ckd, cvp, cvd = _run_prep(
        x3, y3, wq_feat, wq_dir, wkc_f, wkc_d, wvc_f, wvc_d)

    gidx = _run_knn(y_rows, y_cols)

    t0 = y3[:, 0].reshape(B * N, Cc)
    t1 = y3[:, 1].reshape(B * N, Cc)
    t2 = y3[:, 2].reshape(B * N, Cc)
    idx_flat = jnp.transpose(gidx, (0, 2, 1)).reshape(-1)
    g0, g1, g2 = _run_sc_gather(t0, t1, t2, idx_flat)

    out3 = _run_attend(g0, g1, g2, ckp, ckd, cvp, cvd, qx3, x3,
                       wka_f, wka_d, wva_f, wva_d)
    return jnp.transpose(out3, (0, 3, 1, 2))


# speedup vs baseline: 13.0724x; 1.4457x over previous
import functools

import jax
import jax.numpy as jnp
from jax import lax
from jax.experimental import pallas as pl
from jax.experimental.pallas import tpu as pltpu
from jax.experimental.pallas import tpu_sc as plsc

C = 64
K = 16
HEAD_C = 16
NEG = 0.2
ONE_MINUS_NEG = 0.8
EPS = 1e-6
NORM_EPS = 1e-12

NB_PREP = 512
RB_KNN = 512
MB_ATT = 128
SC_CHUNK = 128


def _vn_leaky(p_list, d_list):
    dot = sum(p * d for p, d in zip(p_list, d_list))
    dsq = sum(d * d for d in d_list)
    fac = jnp.where(dot < 0, ONE_MINUS_NEG * dot / (dsq + EPS), 0.0)
    return [p - fac * d for p, d in zip(p_list, d_list)]


def _mm(a, w):
    return lax.dot_general(a, w, (((1,), (1,)), ((), ())),
                           preferred_element_type=jnp.float32)


def _prep_body(x_ref, y_ref, wqf_ref, wqd_ref, wkcf_ref, wkcd_ref,
               wvcf_ref, wvcd_ref,
               qx_ref, ckp_ref, ckd_ref, cvp_ref, cvd_ref):
    wqf, wqd = wqf_ref[...], wqd_ref[...]
    qp = [_mm(x_ref[0, d], wqf) for d in range(3)]
    qd = [_mm(x_ref[0, d], wqd) for d in range(3)]
    q = _vn_leaky(qp, qd)
    qn2 = sum(t * t for t in q)
    tot = jnp.sqrt(jnp.sum(qn2, axis=1, keepdims=True))
    inv_tot = 1.0 / jnp.maximum(tot, NORM_EPS)
    for d in range(3):
        qx_ref[0, d] = q[d] * inv_tot
    for w_ref, o_ref in ((wkcf_ref, ckp_ref), (wkcd_ref, ckd_ref),
                         (wvcf_ref, cvp_ref), (wvcd_ref, cvd_ref)):
        w = w_ref[...]
        for d in range(3):
            o_ref[0, d] = _mm(y_ref[0, d], w)


def _run_prep(x3, y3, wqf, wqd, wkcf, wkcd, wvcf, wvcd):
    B, _, N, _ = x3.shape
    grid = (B, N // NB_PREP)
    blk = pl.BlockSpec((1, 3, NB_PREP, C), lambda b, i: (b, 0, i, 0))
    wblk = pl.BlockSpec((C, C), lambda b, i: (0, 0))
    out = jax.ShapeDtypeStruct((B, 3, N, C), jnp.float32)
    return pl.pallas_call(
        _prep_body,
        grid=grid,
        in_specs=[blk, blk, wblk, wblk, wblk, wblk, wblk, wblk],
        out_specs=[blk] * 5,
        out_shape=[out] * 5,
    )(x3, y3, wqf, wqd, wkcf, wkcd, wvcf, wvcd)


def _knn_body(yr_ref, yc_ref, idx_ref, *, n):
    yr = yr_ref[0]
    yc = yc_ref[0]
    g = lax.dot_general(yr, yc, (((1,), (0,)), ((), ())),
                        preferred_element_type=jnp.float32)
    s_row = jnp.sum(yr * yr, axis=1, keepdims=True)
    s_col = jnp.sum(yc * yc, axis=0, keepdims=True)
    vals = (2.0 * g - s_col) - s_row
    u = lax.bitcast_convert_type(vals, jnp.int32)
    ki = jnp.where(u >= 0, u, u ^ jnp.int32(0x7FFFFFFF))
    col = lax.broadcasted_iota(jnp.int32, vals.shape, 1)
    keys = (ki & jnp.int32(~2047)) | (jnp.int32(n - 1) - col)
    picks = []
    int_min = jnp.int32(-(2 ** 31))
    for _ in range(K):
        m = jnp.max(keys, axis=1, keepdims=True)
        picks.append(jnp.int32(n - 1) - (m & jnp.int32(2047)))
        keys = jnp.where(keys == m, int_min, keys)
    base = pl.program_id(0) * n
    idx_ref[0] = jnp.concatenate(picks, axis=1) + base


def _run_knn(y_rows, y_cols):
    B, N, _ = y_rows.shape
    grid = (B, N // RB_KNN)
    return pl.pallas_call(
        functools.partial(_knn_body, n=N),
        grid=grid,
        in_specs=[
            pl.BlockSpec((1, RB_KNN, 3 * C), lambda b, i: (b, i, 0)),
            pl.BlockSpec((1, 3 * C, N), lambda b, i: (b, 0, 0)),
        ],
        out_specs=pl.BlockSpec((1, RB_KNN, K), lambda b, i: (b, i, 0)),
        out_shape=jax.ShapeDtypeStruct((B, N, K), jnp.int32),
    )(y_rows, y_cols)


def _sc_gather_body(nc, n_chunks, t0, t1, t2, idx_hbm, g0, g1, g2,
                    idx_v, r0, r1, r2, sem0, sem1, sem2):
    wid = lax.axis_index("s") * nc + lax.axis_index("c")
    base = wid * (n_chunks * SC_CHUNK)

    def chunk(i, carry):
        off = pl.multiple_of(base + i * SC_CHUNK, SC_CHUNK)
        pltpu.sync_copy(idx_hbm.at[pl.ds(off, SC_CHUNK)], idx_v)
        cp0 = pltpu.async_copy(t0.at[idx_v], r0, sem0)
        cp1 = pltpu.async_copy(t1.at[idx_v], r1, sem1)
        cp2 = pltpu.async_copy(t2.at[idx_v], r2, sem2)
        cp0.wait()
        pltpu.sync_copy(r0, g0.at[pl.ds(off, SC_CHUNK)])
        cp1.wait()
        pltpu.sync_copy(r1, g1.at[pl.ds(off, SC_CHUNK)])
        cp2.wait()
        pltpu.sync_copy(r2, g2.at[pl.ds(off, SC_CHUNK)])
        return carry

    lax.fori_loop(0, n_chunks, chunk, 0)


def _run_sc_gather(t0, t1, t2, idx_flat):
    (p,) = idx_flat.shape
    info = plsc.get_sparse_core_info()
    nc, ns = info.num_cores, info.num_subcores
    nw = nc * ns
    n_chunks = p // (nw * SC_CHUNK)
    mesh = plsc.VectorSubcoreMesh(core_axis_name="c", subcore_axis_name="s")
    out = jax.ShapeDtypeStruct((p, C), jnp.float32)
    run = pl.kernel(
        functools.partial(_sc_gather_body, nc, n_chunks),
        out_type=[out] * 3,
        mesh=mesh,
        compiler_params=pltpu.CompilerParams(use_tc_tiling_on_sc=False),
        scratch_types=[
            pltpu.VMEM((SC_CHUNK,), jnp.int32),
            pltpu.VMEM((SC_CHUNK, C), jnp.float32),
            pltpu.VMEM((SC_CHUNK, C), jnp.float32),
            pltpu.VMEM((SC_CHUNK, C), jnp.float32),
            pltpu.SemaphoreType.DMA,
            pltpu.SemaphoreType.DMA,
            pltpu.SemaphoreType.DMA,
        ],
    )
    return run(t0, t1, t2, idx_flat)


def _swap64(t):
    return jnp.concatenate((t[..., C:], t[..., :C]), axis=-1)


def _dup(t2d):
    return jnp.concatenate((t2d, t2d), axis=-1).reshape(1, MB_ATT, 2 * C)


def _attend_body(g0_ref, g1_ref, g2_ref, ckp_ref, ckd_ref, cvp_ref,
                 cvd_ref, qx_ref, x_ref, wkaf_ref, wkad_ref, wvaf_ref,
                 wvad_ref, out_ref):
    kh = K // 2
    g_refs = (g0_ref, g1_ref, g2_ref)
    wkaf, wkad = wkaf_ref[...], wkad_ref[...]
    wvaf, wvad = wvaf_ref[...], wvad_ref[...]
    pk, dk, pv, dv = [], [], [], []
    for d in range(3):
        gm = g_refs[d][0, :, 0].reshape(kh * MB_ATT, 2 * C)
        pk.append(_mm(gm, wkaf).reshape(kh, MB_ATT, 2 * C)
                  + _dup(ckp_ref[0, d]))
        dk.append(_mm(gm, wkad).reshape(kh, MB_ATT, 2 * C)
                  + _dup(ckd_ref[0, d]))
        pv.append(_mm(gm, wvaf).reshape(kh, MB_ATT, 2 * C)
                  + _dup(cvp_ref[0, d]))
        dv.append(_mm(gm, wvad).reshape(kh, MB_ATT, 2 * C)
                  + _dup(cvd_ref[0, d]))
    kf = _vn_leaky(pk, dk)
    vf = _vn_leaky(pv, dv)
    qn2 = sum(t * t for t in kf)
    li = lax.broadcasted_iota(jnp.int32, (2 * C, 2 * C), 0)
    lj = lax.broadcasted_iota(jnp.int32, (2 * C, 2 * C), 1)
    seg = (li // C == lj // C).astype(jnp.float32)
    tot2 = lax.dot_general(qn2.reshape(kh * MB_ATT, 2 * C), seg,
                           (((1,), (0,)), ((), ())),
                           preferred_element_type=jnp.float32)
    inv_tot = lax.rsqrt(jnp.maximum(tot2, NORM_EPS * NORM_EPS))
    inv_tot = inv_tot.reshape(kh, MB_ATT, 2 * C)
    qxs = [_dup(qx_ref[0, d]) for d in range(3)]
    qk = sum(kf[d] * qxs[d] for d in range(3)) * inv_tot
    h = (li // HEAD_C == lj // HEAD_C).astype(jnp.float32)
    h = h / jnp.sqrt(jnp.float32(3 * HEAD_C))
    s = lax.dot_general(qk.reshape(kh * MB_ATT, 2 * C), h,
                        (((1,), (0,)), ((), ())),
                        preferred_element_type=jnp.float32)
    s = s.reshape(kh, MB_ATT, 2 * C)
    mx = jnp.max(s, axis=0, keepdims=True)
    mx = jnp.maximum(mx, _swap64(mx))
    e = jnp.exp(s - mx)
    se = jnp.sum(e, axis=0)
    r = 1.0 / (se + _swap64(se))
    for d in range(3):
        acc = jnp.sum(e * vf[d], axis=0) * r
        out_ref[0, d] = x_ref[0, d] + (acc[:, :C] + acc[:, C:])


def _run_attend(g0, g1, g2, ckp, ckd, cvp, cvd, qx3, x3,
                wkaf, wkad, wvaf, wvad):
    B, _, N, _ = x3.shape
    nblk = N // MB_ATT
    grid = (B, nblk)
    gshape = (B, K // 2, nblk, MB_ATT, 2 * C)
    gspec = pl.BlockSpec((1, K // 2, 1, MB_ATT, 2 * C),
                         lambda b, i: (b, 0, i, 0, 0))
    blk = pl.BlockSpec((1, 3, MB_ATT, C), lambda b, i: (b, 0, i, 0))
    wblk = pl.BlockSpec((2 * C, 2 * C), lambda b, i: (0, 0))

    def bd(w):
        z = jnp.zeros((C, C), jnp.float32)
        return jnp.concatenate(
            (jnp.concatenate((w, z), axis=1),
             jnp.concatenate((z, w), axis=1)), axis=0)

    return pl.pallas_call(
        _attend_body,
        grid=grid,
        in_specs=[gspec, gspec, gspec, blk, blk, blk, blk, blk, blk,
                  wblk, wblk, wblk, wblk],
        out_specs=blk,
        out_shape=jax.ShapeDtypeStruct((B, 3, N, C), jnp.float32),
    )(g0.reshape(gshape), g1.reshape(gshape), g2.reshape(gshape),
      ckp, ckd, cvp, cvd, qx3, x3,
      bd(wkaf), bd(wkad), bd(wvaf), bd(wvad))


def kernel(x, y, wq_feat, wq_dir, wk_feat, wk_dir, wv_feat, wv_dir):
    B, Cc, _, N = x.shape
    x3 = jnp.transpose(x, (0, 2, 3, 1))
    y3 = jnp.transpose(y, (0, 2, 3, 1))
    y_rows = jnp.transpose(y, (0, 3, 2, 1)).reshape(B, N, 3 * Cc)
    y_cols = jnp.transpose(y, (0, 2, 1, 3)).reshape(B, 3 * Cc, N)

    wka_f, wkc_f = wk_feat[:, :Cc], wk_feat[:, Cc:] - wk_feat[:, :Cc]
    wka_d, wkc_d = wk_dir[:, :Cc], wk_dir[:, Cc:] - wk_dir[:, :Cc]
    wva_f, wvc_f = wv_feat[:, :Cc], wv_feat[:, Cc:] - wv_feat[:, :Cc]
    wva_d, wvc_d = wv_dir[:, :Cc], wv_dir[:, Cc:] - wv_dir[:, :Cc]

    qx3, ckp, ckd, cvp, cvd = _run_prep(
        x3, y3, wq_feat, wq_dir, wkc_f, wkc_d, wvc_f, wvc_d)

    gidx = _run_knn(y_rows, y_cols)

    t0 = y3[:, 0].reshape(B * N, Cc)
    t1 = y3[:, 1].reshape(B * N, Cc)
    t2 = y3[:, 2].reshape(B * N, Cc)
    idx_flat = jnp.transpose(gidx, (0, 2, 1)).reshape(B, K // 2, 2, N)
    idx_flat = jnp.transpose(idx_flat, (0, 1, 3, 2)).reshape(-1)
    g0, g1, g2 = _run_sc_gather(t0, t1, t2, idx_flat)

    out3 = _run_attend(g0, g1, g2, ckp, ckd, cvp, cvd, qx3, x3,
                       wka_f, wka_d, wva_f, wva_d)
    return jnp.transpose(out3, (0, 3, 1, 2))
